# baseline (speedup 1.0000x reference)
import functools

import jax
import jax.numpy as jnp
from jax import lax
from jax.experimental import pallas as pl
from jax.experimental.pallas import tpu as pltpu
from jax.experimental.pallas import tpu_sc as plsc

N_NODES = 10000
N_EDGES = 320000
D = 128

NC = 2
NS = 16
NW = NC * NS
K = 128
CHUNKS = 78
EPW = CHUNKS * K
TAIL_EDGE_BASE = NW * EPW
TAIL_CHUNKS = (N_EDGES - TAIL_EDGE_BASE) // K
HALF0 = 40
HALF1 = CHUNKS - HALF0
ROWS_PER_SUB = 624
TAIL_BASE = NS * ROWS_PER_SUB
TAIL_ROWS = N_NODES - TAIL_BASE

ROW_BLK = 10000


def _prologue_body(feat_ref, cj_ref, w_ref, h_ref):
    h_ref[...] = jnp.dot(
        feat_ref[...] * cj_ref[...], w_ref[...],
        preferred_element_type=jnp.float32)


def _epilogue_body(p_ref, ci_ref, o_ref):
    o_ref[...] = (p_ref[0] + p_ref[1]) * ci_ref[...]


def _sc_agg_body(h_hbm, ei_hbm, out_hbm,
                 src_v, dst_v, rows0, rows1, acc_sh, sem0, sem1):
    cid = lax.axis_index("c")
    sid = lax.axis_index("s")
    wid = cid * NS + sid

    @pl.loop(0, 120)
    def _(r):
        @pl.loop(0, D // 16)
        def _(l):
            rows0[r, pl.ds(l * 16, 16)] = jnp.zeros((16,), jnp.float32)

    zbase = sid * ROWS_PER_SUB
    for t in range(5):
        pltpu.sync_copy(rows0.at[pl.ds(0, 120)],
                        acc_sh.at[pl.ds(zbase + t * 120, 120)])
    pltpu.sync_copy(rows0.at[pl.ds(0, 24)],
                    acc_sh.at[pl.ds(zbase + 600, 24)])

    @pl.when(sid == NS - 1)
    def _():
        pltpu.sync_copy(rows0.at[pl.ds(0, TAIL_ROWS)],
                        acc_sh.at[pl.ds(TAIL_BASE, TAIL_ROWS)])

    plsc.subcore_barrier()

    for nch, estart in ((HALF0, 0), (HALF1, HALF0 * K)):
        base = wid * EPW + estart
        pltpu.sync_copy(ei_hbm.at[0].at[pl.ds(base, nch * K)],
                        src_v.at[pl.ds(0, nch * K)])
        pltpu.sync_copy(ei_hbm.at[1].at[pl.ds(base, nch * K)],
                        dst_v.at[pl.ds(0, nch * K)])

        pltpu.async_copy(h_hbm.at[src_v.at[pl.ds(0, K)]], rows0, sem0)
        pltpu.async_copy(h_hbm.at[src_v.at[pl.ds(K, K)]], rows1, sem1)

        @pl.loop(0, nch, step=2)
        def _(c):
            pltpu.make_async_copy(
                h_hbm.at[src_v.at[pl.ds(c * K, K)]], rows0, sem0).wait()
            pltpu.sync_copy(rows0, acc_sh.at[dst_v.at[pl.ds(c * K, K)]],
                            add=True)

            @pl.when(c + 2 < nch)
            def _():
                pltpu.async_copy(
                    h_hbm.at[src_v.at[pl.ds((c + 2) * K, K)]], rows0, sem0)

            pltpu.make_async_copy(
                h_hbm.at[src_v.at[pl.ds((c + 1) * K, K)]], rows1,
                sem1).wait()
            pltpu.sync_copy(rows1, acc_sh.at[dst_v.at[pl.ds((c + 1) * K, K)]],
                            add=True)

            @pl.when(c + 3 < nch)
            def _():
                pltpu.async_copy(
                    h_hbm.at[src_v.at[pl.ds((c + 3) * K, K)]], rows1, sem1)

    @pl.when(sid < TAIL_CHUNKS // NC)
    def _():
        tbase = TAIL_EDGE_BASE + (cid * (TAIL_CHUNKS // NC) + sid) * K
        pltpu.sync_copy(ei_hbm.at[0].at[pl.ds(tbase, K)],
                        src_v.at[pl.ds(0, K)])
        pltpu.sync_copy(ei_hbm.at[1].at[pl.ds(tbase, K)],
                        dst_v.at[pl.ds(0, K)])
        pltpu.async_copy(h_hbm.at[src_v.at[pl.ds(0, K)]], rows0, sem0).wait()
        pltpu.sync_copy(rows0, acc_sh.at[dst_v.at[pl.ds(0, K)]], add=True)

    plsc.subcore_barrier()

    pltpu.sync_copy(acc_sh.at[pl.ds(zbase, ROWS_PER_SUB)],
                    out_hbm.at[cid].at[pl.ds(zbase, ROWS_PER_SUB)])

    @pl.when(sid == NS - 1)
    def _():
        pltpu.sync_copy(acc_sh.at[pl.ds(TAIL_BASE, TAIL_ROWS)],
                        out_hbm.at[cid].at[pl.ds(TAIL_BASE, TAIL_ROWS)])


@jax.jit
def kernel(feat, edge_index, cj, ci, W):
    ei = edge_index if edge_index.dtype == jnp.int32 else edge_index.astype(
        jnp.int32)

    h = pl.pallas_call(
        _prologue_body,
        grid=(N_NODES // ROW_BLK,),
        in_specs=[
            pl.BlockSpec((ROW_BLK, D), lambda i: (i, 0)),
            pl.BlockSpec((ROW_BLK, 1), lambda i: (i, 0)),
            pl.BlockSpec((D, D), lambda i: (0, 0)),
        ],
        out_specs=pl.BlockSpec((ROW_BLK, D), lambda i: (i, 0)),
        out_shape=jax.ShapeDtypeStruct((N_NODES, D), jnp.float32),
    )(feat, cj, W)

    sc_agg = pl.kernel(
        _sc_agg_body,
        out_type=jax.ShapeDtypeStruct((NC, N_NODES, D), jnp.float32),
        mesh=plsc.VectorSubcoreMesh(core_axis_name="c", subcore_axis_name="s"),
        scratch_types=[
            pltpu.VMEM((HALF0 * K,), jnp.int32),
            pltpu.VMEM((HALF0 * K,), jnp.int32),
            pltpu.VMEM((K, D), jnp.float32),
            pltpu.VMEM((K, D), jnp.float32),
            pltpu.VMEM_SHARED((N_NODES, D), jnp.float32),
            pltpu.SemaphoreType.DMA,
            pltpu.SemaphoreType.DMA,
        ],
    )
    partials = sc_agg(h, ei)

    rst = pl.pallas_call(
        _epilogue_body,
        grid=(N_NODES // ROW_BLK,),
        in_specs=[
            pl.BlockSpec((NC, ROW_BLK, D), lambda i: (0, i, 0)),
            pl.BlockSpec((ROW_BLK, 1), lambda i: (i, 0)),
        ],
        out_specs=pl.BlockSpec((ROW_BLK, D), lambda i: (i, 0)),
        out_shape=jax.ShapeDtypeStruct((N_NODES, D), jnp.float32),
    )(partials, ci)
    return rst


# speedup vs baseline: 1.0105x; 1.0105x over previous
import functools

import jax
import jax.numpy as jnp
from jax import lax
from jax.experimental import pallas as pl
from jax.experimental.pallas import tpu as pltpu
from jax.experimental.pallas import tpu_sc as plsc

N_NODES = 10000
N_EDGES = 320000
D = 128

NC = 2
NS = 16
NW = NC * NS
K = 128
CHUNKS = 78
EPW = CHUNKS * K
TAIL_EDGE_BASE = NW * EPW
TAIL_CHUNKS = (N_EDGES - TAIL_EDGE_BASE) // K
HALF0 = 48
HALF1 = CHUNKS - HALF0
ROWS_PER_SUB = 624
TAIL_BASE = NS * ROWS_PER_SUB
TAIL_ROWS = N_NODES - TAIL_BASE

ROW_BLK = 5000


def _prologue_body(feat_ref, cj_ref, w_ref, h_ref):
    h_ref[...] = jnp.dot(
        feat_ref[...] * cj_ref[...], w_ref[...],
        preferred_element_type=jnp.float32)


def _epilogue_body(p_ref, ci_ref, o_ref):
    o_ref[...] = (p_ref[0] + p_ref[1]) * ci_ref[...]


def _sc_agg_body(h_hbm, ei_hbm, out_hbm,
                 src_v, dst_v, rows0, rows1, acc_sh, sem0, sem1):
    cid = lax.axis_index("c")
    sid = lax.axis_index("s")
    wid = cid * NS + sid

    @pl.loop(0, 120)
    def _(r):
        @pl.loop(0, D // 16)
        def _(l):
            rows0[r, pl.ds(l * 16, 16)] = jnp.zeros((16,), jnp.float32)

    zbase = sid * ROWS_PER_SUB
    for t in range(5):
        pltpu.sync_copy(rows0.at[pl.ds(0, 120)],
                        acc_sh.at[pl.ds(zbase + t * 120, 120)])
    pltpu.sync_copy(rows0.at[pl.ds(0, 24)],
                    acc_sh.at[pl.ds(zbase + 600, 24)])

    @pl.when(sid == NS - 1)
    def _():
        pltpu.sync_copy(rows0.at[pl.ds(0, TAIL_ROWS)],
                        acc_sh.at[pl.ds(TAIL_BASE, TAIL_ROWS)])

    plsc.subcore_barrier()

    for nch, estart in ((HALF0, 0), (HALF1, HALF0 * K)):
        base = wid * EPW + estart
        pltpu.sync_copy(ei_hbm.at[0].at[pl.ds(base, nch * K)],
                        src_v.at[pl.ds(0, nch * K)])
        pltpu.sync_copy(ei_hbm.at[1].at[pl.ds(base, nch * K)],
                        dst_v.at[pl.ds(0, nch * K)])

        pltpu.async_copy(h_hbm.at[src_v.at[pl.ds(0, K)]], rows0, sem0)
        pltpu.async_copy(h_hbm.at[src_v.at[pl.ds(K, K)]], rows1, sem1)

        @pl.loop(0, nch, step=2)
        def _(c):
            pltpu.make_async_copy(
                h_hbm.at[src_v.at[pl.ds(c * K, K)]], rows0, sem0).wait()
            pltpu.sync_copy(rows0, acc_sh.at[dst_v.at[pl.ds(c * K, K)]],
                            add=True)

            @pl.when(c + 2 < nch)
            def _():
                pltpu.async_copy(
                    h_hbm.at[src_v.at[pl.ds((c + 2) * K, K)]], rows0, sem0)

            pltpu.make_async_copy(
                h_hbm.at[src_v.at[pl.ds((c + 1) * K, K)]], rows1,
                sem1).wait()
            pltpu.sync_copy(rows1, acc_sh.at[dst_v.at[pl.ds((c + 1) * K, K)]],
                            add=True)

            @pl.when(c + 3 < nch)
            def _():
                pltpu.async_copy(
                    h_hbm.at[src_v.at[pl.ds((c + 3) * K, K)]], rows1, sem1)

    @pl.when(sid < TAIL_CHUNKS // NC)
    def _():
        tbase = TAIL_EDGE_BASE + (cid * (TAIL_CHUNKS // NC) + sid) * K
        pltpu.sync_copy(ei_hbm.at[0].at[pl.ds(tbase, K)],
                        src_v.at[pl.ds(0, K)])
        pltpu.sync_copy(ei_hbm.at[1].at[pl.ds(tbase, K)],
                        dst_v.at[pl.ds(0, K)])
        pltpu.async_copy(h_hbm.at[src_v.at[pl.ds(0, K)]], rows0, sem0).wait()
        pltpu.sync_copy(rows0, acc_sh.at[dst_v.at[pl.ds(0, K)]], add=True)

    plsc.subcore_barrier()

    pltpu.sync_copy(acc_sh.at[pl.ds(zbase, ROWS_PER_SUB)],
                    out_hbm.at[cid].at[pl.ds(zbase, ROWS_PER_SUB)])

    @pl.when(sid == NS - 1)
    def _():
        pltpu.sync_copy(acc_sh.at[pl.ds(TAIL_BASE, TAIL_ROWS)],
                        out_hbm.at[cid].at[pl.ds(TAIL_BASE, TAIL_ROWS)])


@jax.jit
def kernel(feat, edge_index, cj, ci, W):
    ei = edge_index if edge_index.dtype == jnp.int32 else edge_index.astype(
        jnp.int32)

    h = pl.pallas_call(
        _prologue_body,
        grid=(N_NODES // ROW_BLK,),
        in_specs=[
            pl.BlockSpec((ROW_BLK, D), lambda i: (i, 0)),
            pl.BlockSpec((ROW_BLK, 1), lambda i: (i, 0)),
            pl.BlockSpec((D, D), lambda i: (0, 0)),
        ],
        out_specs=pl.BlockSpec((ROW_BLK, D), lambda i: (i, 0)),
        out_shape=jax.ShapeDtypeStruct((N_NODES, D), jnp.float32),
    )(feat, cj, W)

    sc_agg = pl.kernel(
        _sc_agg_body,
        out_type=jax.ShapeDtypeStruct((NC, N_NODES, D), jnp.float32),
        mesh=plsc.VectorSubcoreMesh(core_axis_name="c", subcore_axis_name="s"),
        scratch_types=[
            pltpu.VMEM((HALF0 * K,), jnp.int32),
            pltpu.VMEM((HALF0 * K,), jnp.int32),
            pltpu.VMEM((K, D), jnp.float32),
            pltpu.VMEM((K, D), jnp.float32),
            pltpu.VMEM_SHARED((N_NODES, D), jnp.float32),
            pltpu.SemaphoreType.DMA,
            pltpu.SemaphoreType.DMA,
        ],
    )
    partials = sc_agg(h, ei)

    rst = pl.pallas_call(
        _epilogue_body,
        grid=(N_NODES // ROW_BLK,),
        in_specs=[
            pl.BlockSpec((NC, ROW_BLK, D), lambda i: (0, i, 0)),
            pl.BlockSpec((ROW_BLK, 1), lambda i: (i, 0)),
        ],
        out_specs=pl.BlockSpec((ROW_BLK, D), lambda i: (i, 0)),
        out_shape=jax.ShapeDtypeStruct((N_NODES, D), jnp.float32),
    )(partials, ci)
    return rst
